# Initial kernel scaffold; baseline (speedup 1.0000x reference)
import jax
import jax.numpy as jnp
from jax.experimental import pallas as pl


def kernel(x, params, edge_index, batch):
    raise NotImplementedError("write your pallas kernel here")


# speedup vs baseline: 7.0949x; 7.0949x over previous
import functools
import math

import jax
import jax.numpy as jnp
from jax import lax
from jax.experimental import pallas as pl
from jax.experimental.pallas import tpu as pltpu
from jax.experimental.pallas import tpu_sc as plsc

N = 10000
E = 320000
DIM = 128
NSEG = 64
NLAYERS = 4

NC = 2
NS = 16
NW = NC * NS
EPW = E // NW
CHUNK = 80
NCHUNK = EPW // CHUNK
NPAD = 10112
ROWS_PER_TILE = NPAD // NS

BLK = 1000
GRID = N // BLK

_BN_RSQRT = 1.0 / math.sqrt(1.0 + 1e-5)


def _agg_body(h_hbm, src_hbm, dst_hbm, out_hbm, srcv, dstv, buf, aggsh):
    c = lax.axis_index("c")
    s = lax.axis_index("s")
    wid = c * NS + s

    def zelem(t, carry):
        buf[t // 8, pl.ds((t % 8) * 16, 16)] = jnp.zeros((16,), jnp.float32)
        return carry

    lax.fori_loop(0, CHUNK * 8, zelem, 0)
    base = s * ROWS_PER_TILE
    for k in range(7):
        pltpu.sync_copy(buf, aggsh.at[pl.ds(base + k * CHUNK, CHUNK)])
    pltpu.sync_copy(buf.at[pl.ds(0, 72)], aggsh.at[pl.ds(base + 560, 72)])
    plsc.subcore_barrier()

    pltpu.sync_copy(src_hbm.at[wid], srcv)
    pltpu.sync_copy(dst_hbm.at[wid], dstv)

    def body(i, carry):
        pltpu.sync_copy(h_hbm.at[srcv.at[i]], buf)
        pltpu.sync_copy(buf, aggsh.at[dstv.at[i]], add=True)
        return carry

    lax.fori_loop(0, NCHUNK, body, 0)
    plsc.subcore_barrier()

    pltpu.sync_copy(
        aggsh.at[pl.ds(s * ROWS_PER_TILE, ROWS_PER_TILE)],
        out_hbm.at[c, pl.ds(s * ROWS_PER_TILE, ROWS_PER_TILE)],
    )


@functools.cache
def _make_agg():
    return pl.kernel(
        _agg_body,
        mesh=plsc.VectorSubcoreMesh(core_axis_name="c", subcore_axis_name="s"),
        out_type=jax.ShapeDtypeStruct((NC, NPAD, DIM), jnp.float32),
        scratch_types=[
            pltpu.VMEM((NCHUNK, CHUNK), jnp.int32),
            pltpu.VMEM((NCHUNK, CHUNK), jnp.int32),
            pltpu.VMEM((CHUNK, DIM), jnp.float32),
            pltpu.VMEM_SHARED((NPAD, DIM), jnp.float32),
        ],
    )


def _mlp_math(eps_ref, h_ref, a0_ref, a1_ref, w1_ref, b1_ref, w2_ref, b2_ref,
              g_ref, bb_ref):
    h = h_ref[...]
    z = (1.0 + eps_ref[0, 0]) * h + a0_ref[...] + a1_ref[...]
    z = jnp.maximum(
        jnp.dot(z, w1_ref[...], preferred_element_type=jnp.float32) + b1_ref[...],
        0.0)
    z = jnp.dot(z, w2_ref[...], preferred_element_type=jnp.float32) + b2_ref[...]
    z = g_ref[...] * (z * _BN_RSQRT) + bb_ref[...]
    return h, jnp.maximum(z, 0.0)


def _onehot(batch_ref):
    seg = lax.broadcasted_iota(jnp.int32, (NSEG, BLK), 0)
    return (seg == batch_ref[0]).astype(jnp.float32)


def _mlp_body(eps_ref, h_ref, a0_ref, a1_ref, w1_ref, b1_ref, w2_ref, b2_ref,
              g_ref, bb_ref, batch_ref, hout_ref, pool_ref):
    h, h1 = _mlp_math(eps_ref, h_ref, a0_ref, a1_ref, w1_ref, b1_ref, w2_ref,
                      b2_ref, g_ref, bb_ref)
    hout_ref[...] = h1
    oh = _onehot(batch_ref)

    @pl.when(pl.program_id(0) == 0)
    def _():
        pool_ref[...] = jnp.zeros_like(pool_ref)

    pool_ref[...] += jnp.dot(oh, h1, preferred_element_type=jnp.float32)


def _mlp_body_poolin(eps_ref, h_ref, a0_ref, a1_ref, w1_ref, b1_ref, w2_ref,
                     b2_ref, g_ref, bb_ref, batch_ref, hout_ref, pool_ref,
                     poolx_ref):
    h, h1 = _mlp_math(eps_ref, h_ref, a0_ref, a1_ref, w1_ref, b1_ref, w2_ref,
                      b2_ref, g_ref, bb_ref)
    hout_ref[...] = h1
    oh = _onehot(batch_ref)

    @pl.when(pl.program_id(0) == 0)
    def _():
        pool_ref[...] = jnp.zeros_like(pool_ref)
        poolx_ref[...] = jnp.zeros_like(poolx_ref)

    pool_ref[...] += jnp.dot(oh, h1, preferred_element_type=jnp.float32)
    poolx_ref[...] += jnp.dot(oh, h, preferred_element_type=jnp.float32)


def _row_spec():
    return pl.BlockSpec((BLK, DIM), lambda i: (i, 0))


def _full_spec(shape):
    nd = len(shape)
    return pl.BlockSpec(shape, lambda i: (0,) * nd)


_MLP_IN_SPECS = [
    pl.BlockSpec(memory_space=pltpu.SMEM),
    _row_spec(),
    _row_spec(),
    _row_spec(),
    _full_spec((DIM, DIM)),
    _full_spec((1, DIM)),
    _full_spec((DIM, DIM)),
    _full_spec((1, DIM)),
    _full_spec((1, DIM)),
    _full_spec((1, DIM)),
    pl.BlockSpec((1, 1, BLK), lambda i: (i, 0, 0)),
]

_mlp_call = pl.pallas_call(
    _mlp_body,
    grid=(GRID,),
    in_specs=_MLP_IN_SPECS,
    out_specs=[_row_spec(), _full_spec((NSEG, DIM))],
    out_shape=[
        jax.ShapeDtypeStruct((N, DIM), jnp.float32),
        jax.ShapeDtypeStruct((NSEG, DIM), jnp.float32),
    ],
)

_mlp_call_poolin = pl.pallas_call(
    _mlp_body_poolin,
    grid=(GRID,),
    in_specs=_MLP_IN_SPECS,
    out_specs=[_row_spec(), _full_spec((NSEG, DIM)), _full_spec((NSEG, DIM))],
    out_shape=[
        jax.ShapeDtypeStruct((N, DIM), jnp.float32),
        jax.ShapeDtypeStruct((NSEG, DIM), jnp.float32),
        jax.ShapeDtypeStruct((NSEG, DIM), jnp.float32),
    ],
)


def _pred_body(gemb_ref, w_ref, b_ref, out_ref):
    out_ref[...] = (
        jnp.dot(gemb_ref[...], w_ref[...], preferred_element_type=jnp.float32)
        + b_ref[...])


_PRED_DIM = DIM + NLAYERS * DIM

_pred_call = pl.pallas_call(
    _pred_body,
    grid=(1,),
    in_specs=[
        _full_spec((NSEG, _PRED_DIM)),
        _full_spec((_PRED_DIM, DIM)),
        _full_spec((1, DIM)),
    ],
    out_specs=_full_spec((NSEG, DIM)),
    out_shape=jax.ShapeDtypeStruct((NSEG, DIM), jnp.float32),
)


def kernel(x, params, edge_index, batch):
    src2d = edge_index[0].reshape(NW, NCHUNK, CHUNK)
    dst2d = edge_index[1].reshape(NW, NCHUNK, CHUNK)
    batch3d = batch.reshape(GRID, 1, BLK)

    h = x
    pools = []
    for l in range(NLAYERS):
        agg = _make_agg()(h, src2d, dst2d)[:, :N, :]
        args = (
            params["eps_%d" % l].reshape(1, 1),
            h, agg[0], agg[1],
            params["W1_%d" % l], params["b1_%d" % l].reshape(1, DIM),
            params["W2_%d" % l], params["b2_%d" % l].reshape(1, DIM),
            params["bn_g_%d" % l].reshape(1, DIM),
            params["bn_b_%d" % l].reshape(1, DIM),
            batch3d,
        )
        if l == 0:
            h, p, px = _mlp_call_poolin(*args)
            pools = [px, p]
        else:
            h, p = _mlp_call(*args)
            pools.append(p)

    gemb = jnp.concatenate(pools, axis=1)
    return _pred_call(gemb, params["W_pred"], params["b_pred"].reshape(1, DIM))
